# Initial kernel scaffold; baseline (speedup 1.0000x reference)
import functools

import jax
import jax.numpy as jnp
from jax import lax
from jax.experimental import pallas as pl
from jax.experimental.pallas import tpu as pltpu
from jax.experimental.pallas import tpu_sc as plsc

VOCAB = 100000
EMBED = 1024
MAXLEN = 2048
BATCH = 4
SEQ = 2048

NUM_WORKERS = 32
ROWS_PER_W = (BATCH * SEQ) // NUM_WORKERS
CHUNK = 32
NCHUNKS = ROWS_PER_W // CHUNK


def _emb_kernel(tok_hbm, ids_hbm, pos_hbm, out_hbm, idx_v, buf_v, sem):
    nc = 2
    wid = lax.axis_index("s") * nc + lax.axis_index("c")
    base = wid * ROWS_PER_W
    s0 = lax.rem(base, SEQ)

    pltpu.sync_copy(ids_hbm.at[pl.ds(base, ROWS_PER_W)], idx_v)

    for i in range(NCHUNKS):
        off = i * CHUNK
        pltpu.sync_copy(pos_hbm.at[pl.ds(s0 + off, CHUNK)], buf_v)
        pltpu.async_copy(
            tok_hbm.at[idx_v.at[pl.ds(off, CHUNK)]], buf_v, sem, add=True
        ).wait()
        pltpu.sync_copy(buf_v, out_hbm.at[pl.ds(base + off, CHUNK)])


@jax.jit
def _embedding(ids_flat, token_table, pos_table):
    mesh = plsc.VectorSubcoreMesh(core_axis_name="c", subcore_axis_name="s")
    k = functools.partial(
        pl.kernel,
        mesh=mesh,
        out_type=jax.ShapeDtypeStruct((BATCH * SEQ, EMBED), jnp.float32),
        scratch_types=[
            pltpu.VMEM((ROWS_PER_W,), jnp.int32),
            pltpu.VMEM((CHUNK, EMBED), jnp.float32),
            pltpu.SemaphoreType.DMA,
        ],
    )(_emb_kernel)
    return k(token_table, ids_flat, pos_table)


def kernel(input_ids, token_table, pos_table):
    batch, seq = input_ids.shape
    ids_flat = input_ids.reshape(batch * seq).astype(jnp.int32)
    out = _embedding(ids_flat, token_table, pos_table)
    return out.reshape(batch, seq, EMBED)


# speedup vs baseline: 1.2071x; 1.2071x over previous
import functools

import jax
import jax.numpy as jnp
from jax import lax
from jax.experimental import pallas as pl
from jax.experimental.pallas import tpu as pltpu
from jax.experimental.pallas import tpu_sc as plsc

VOCAB = 100000
EMBED = 1024
MAXLEN = 2048
BATCH = 4
SEQ = 2048

NUM_WORKERS = 32
ROWS_PER_W = (BATCH * SEQ) // NUM_WORKERS
CHUNK = 32
NCHUNKS = ROWS_PER_W // CHUNK


def _emb_kernel(tok_hbm, ids_hbm, pos_hbm, out_hbm, idx_v, buf_v, pos_v, sem):
    nc = 2
    wid = lax.axis_index("s") * nc + lax.axis_index("c")
    base = wid * ROWS_PER_W
    s0 = lax.rem(base, SEQ)

    pltpu.sync_copy(ids_hbm.at[pl.ds(base, ROWS_PER_W)], idx_v)

    for i in range(NCHUNKS):
        off = i * CHUNK
        gather = pltpu.async_copy(
            tok_hbm.at[idx_v.at[pl.ds(off, CHUNK)]], buf_v, sem
        )
        pltpu.sync_copy(pos_hbm.at[pl.ds(s0 + off, CHUNK)], pos_v)
        gather.wait()

        def add_row(r):
            for j in range(EMBED // 16):
                plsc.addupdate(
                    buf_v.at[r, pl.ds(j * 16, 16)], pos_v[r, pl.ds(j * 16, 16)]
                )

        pl.loop(0, CHUNK)(add_row)

        pltpu.sync_copy(buf_v, out_hbm.at[pl.ds(base + off, CHUNK)])


@jax.jit
def _embedding(ids_flat, token_table, pos_table):
    mesh = plsc.VectorSubcoreMesh(core_axis_name="c", subcore_axis_name="s")
    k = functools.partial(
        pl.kernel,
        mesh=mesh,
        out_type=jax.ShapeDtypeStruct((BATCH * SEQ, EMBED), jnp.float32),
        scratch_types=[
            pltpu.VMEM((ROWS_PER_W,), jnp.int32),
            pltpu.VMEM((CHUNK, EMBED), jnp.float32),
            pltpu.VMEM((CHUNK, EMBED), jnp.float32),
            pltpu.SemaphoreType.DMA,
        ],
    )(_emb_kernel)
    return k(token_table, ids_flat, pos_table)


def kernel(input_ids, token_table, pos_table):
    batch, seq = input_ids.shape
    ids_flat = input_ids.reshape(batch * seq).astype(jnp.int32)
    out = _embedding(ids_flat, token_table, pos_table)
    return out.reshape(batch, seq, EMBED)


# speedup vs baseline: 1.4530x; 1.2037x over previous
import functools

import jax
import jax.numpy as jnp
from jax import lax
from jax.experimental import pallas as pl
from jax.experimental.pallas import tpu as pltpu
from jax.experimental.pallas import tpu_sc as plsc

VOCAB = 100000
EMBED = 1024
MAXLEN = 2048
BATCH = 4
SEQ = 2048

NUM_WORKERS = 32
ROWS_PER_W = (BATCH * SEQ) // NUM_WORKERS
CHUNK = 16
NCHUNKS = ROWS_PER_W // CHUNK
NBUF = 3


def _emb_kernel(
    tok_hbm, ids_hbm, pos_hbm, out_hbm, idx_v, tok_v, pos_v, sem_g, sem_p, sem_o
):
    nc = 2
    wid = lax.axis_index("s") * nc + lax.axis_index("c")
    base = wid * ROWS_PER_W
    s0 = lax.rem(base, SEQ)

    pltpu.sync_copy(ids_hbm.at[pl.ds(base, ROWS_PER_W)], idx_v)

    def start_chunk(i):
        b = i % NBUF
        off = i * CHUNK
        g = pltpu.async_copy(
            tok_hbm.at[idx_v.at[pl.ds(off, CHUNK)]], tok_v.at[b], sem_g.at[b]
        )
        p = pltpu.async_copy(
            pos_hbm.at[pl.ds(s0 + off, CHUNK)], pos_v.at[b], sem_p.at[b]
        )
        return g, p

    def finish_chunk(i, g, p):
        b = i % NBUF
        off = i * CHUNK
        g.wait()
        p.wait()

        def add_row(r):
            for j in range(EMBED // 16):
                plsc.addupdate(
                    tok_v.at[b, r, pl.ds(j * 16, 16)],
                    pos_v[b, r, pl.ds(j * 16, 16)],
                )

        pl.loop(0, CHUNK)(add_row)
        return pltpu.async_copy(
            tok_v.at[b], out_hbm.at[pl.ds(base + off, CHUNK)], sem_o.at[b]
        )

    inflight = {}
    out_cp = {}
    for i in range(NCHUNKS + 1):
        if i < NCHUNKS:
            if i >= NBUF:
                out_cp.pop(i - NBUF).wait()
            inflight[i] = start_chunk(i)
        j = i - 1
        if j >= 0:
            g, p = inflight.pop(j)
            out_cp[j] = finish_chunk(j, g, p)
    for j in sorted(out_cp):
        out_cp.pop(j).wait()


@jax.jit
def _embedding(ids_flat, token_table, pos_table):
    mesh = plsc.VectorSubcoreMesh(core_axis_name="c", subcore_axis_name="s")
    k = functools.partial(
        pl.kernel,
        mesh=mesh,
        out_type=jax.ShapeDtypeStruct((BATCH * SEQ, EMBED), jnp.float32),
        scratch_types=[
            pltpu.VMEM((ROWS_PER_W,), jnp.int32),
            pltpu.VMEM((NBUF, CHUNK, EMBED), jnp.float32),
            pltpu.VMEM((NBUF, CHUNK, EMBED), jnp.float32),
            pltpu.SemaphoreType.DMA((NBUF,)),
            pltpu.SemaphoreType.DMA((NBUF,)),
            pltpu.SemaphoreType.DMA((NBUF,)),
        ],
    )(_emb_kernel)
    return k(token_table, ids_flat, pos_table)


def kernel(input_ids, token_table, pos_table):
    batch, seq = input_ids.shape
    ids_flat = input_ids.reshape(batch * seq).astype(jnp.int32)
    out = _embedding(ids_flat, token_table, pos_table)
    return out.reshape(batch, seq, EMBED)


# speedup vs baseline: 1.4972x; 1.0304x over previous
import functools

import jax
import jax.numpy as jnp
from jax import lax
from jax.experimental import pallas as pl
from jax.experimental.pallas import tpu as pltpu
from jax.experimental.pallas import tpu_sc as plsc

VOCAB = 100000
EMBED = 1024
MAXLEN = 2048
BATCH = 4
SEQ = 2048

NUM_WORKERS = 32
ROWS_PER_W = (BATCH * SEQ) // NUM_WORKERS
CHUNK = 16
NCHUNKS = ROWS_PER_W // CHUNK
NBUF = 4
NBUF_P = 3
LAG = 2


def _emb_kernel(
    tok_hbm, ids_hbm, pos_hbm, out_hbm, idx_v, tok_v, pos_v, sem_g, sem_p, sem_o
):
    nc = 2
    wid = lax.axis_index("s") * nc + lax.axis_index("c")
    base = wid * ROWS_PER_W
    s0 = lax.rem(base, SEQ)

    pltpu.sync_copy(ids_hbm.at[pl.ds(base, ROWS_PER_W)], idx_v)

    def start_chunk(i):
        b = i % NBUF
        bp = i % NBUF_P
        off = i * CHUNK
        g = pltpu.async_copy(
            tok_hbm.at[idx_v.at[pl.ds(off, CHUNK)]], tok_v.at[b], sem_g.at[b]
        )
        p = pltpu.async_copy(
            pos_hbm.at[pl.ds(s0 + off, CHUNK)], pos_v.at[bp], sem_p.at[bp]
        )
        return g, p

    def finish_chunk(i, g, p):
        b = i % NBUF
        bp = i % NBUF_P
        off = i * CHUNK
        g.wait()
        p.wait()

        def add_row(r):
            for j in range(EMBED // 16):
                plsc.addupdate(
                    tok_v.at[b, r, pl.ds(j * 16, 16)],
                    pos_v[bp, r, pl.ds(j * 16, 16)],
                )

        pl.loop(0, CHUNK)(add_row)
        return pltpu.async_copy(
            tok_v.at[b], out_hbm.at[pl.ds(base + off, CHUNK)], sem_o.at[b]
        )

    inflight = {}
    out_cp = {}
    for i in range(NCHUNKS + LAG):
        if i < NCHUNKS:
            if i >= NBUF:
                out_cp.pop(i - NBUF).wait()
            inflight[i] = start_chunk(i)
        j = i - LAG
        if j >= 0:
            g, p = inflight.pop(j)
            out_cp[j] = finish_chunk(j, g, p)
    for j in sorted(out_cp):
        out_cp.pop(j).wait()


@jax.jit
def _embedding(ids_flat, token_table, pos_table):
    mesh = plsc.VectorSubcoreMesh(core_axis_name="c", subcore_axis_name="s")
    k = functools.partial(
        pl.kernel,
        mesh=mesh,
        out_type=jax.ShapeDtypeStruct((BATCH * SEQ, EMBED), jnp.float32),
        scratch_types=[
            pltpu.VMEM((ROWS_PER_W,), jnp.int32),
            pltpu.VMEM((NBUF, CHUNK, EMBED), jnp.float32),
            pltpu.VMEM((NBUF_P, CHUNK, EMBED), jnp.float32),
            pltpu.SemaphoreType.DMA((NBUF,)),
            pltpu.SemaphoreType.DMA((NBUF_P,)),
            pltpu.SemaphoreType.DMA((NBUF,)),
        ],
    )(_emb_kernel)
    return k(token_table, ids_flat, pos_table)


def kernel(input_ids, token_table, pos_table):
    batch, seq = input_ids.shape
    ids_flat = input_ids.reshape(batch * seq).astype(jnp.int32)
    out = _embedding(ids_flat, token_table, pos_table)
    return out.reshape(batch, seq, EMBED)
